# baseline (speedup 1.0000x reference)
import jax
import jax.numpy as jnp
from jax import lax
from jax.experimental import pallas as pl
from jax.experimental.pallas import tpu as pltpu
from jax.experimental.pallas import tpu_sc as plsc

B = 4096
L = 50
NF = 5
CONT = 8
LARGE_DIM = 64
SMALL_DIM = 16
LANES = 16
SMALL_VOCAB = 100000

NC = 2
NS = 16
W = NC * NS
UPW = B // W
SPW = UPW * L
CH = 128
NCH = SPW // CH
GR = CH // LANES


def _sc_kernel(iidx_hbm, icont_hbm, uidx_hbm, ucont_hbm, tb, ts,
               out_hbm,
               idx_v, uidx_v, u0v, u1v, u2v, u3v, u4v, ucontv,
               rows0, rows1, rows2, rows3, rows4, icontv,
               accb, outv, sem):
    wid = lax.axis_index("s") * NC + lax.axis_index("c")

    pltpu.sync_copy(iidx_hbm.at[wid], idx_v)
    pltpu.sync_copy(uidx_hbm.at[wid], uidx_v)
    pltpu.sync_copy(ucont_hbm.at[wid], ucontv)

    pltpu.async_copy(tb.at[uidx_v.at[0]], u0v, sem).wait()
    pltpu.async_copy(ts.at[uidx_v.at[1]], u1v, sem).wait()
    pltpu.async_copy(ts.at[uidx_v.at[2]], u2v, sem).wait()
    pltpu.async_copy(ts.at[uidx_v.at[3]], u3v, sem).wait()
    pltpu.async_copy(ts.at[uidx_v.at[4]], u4v, sem).wait()

    def chunk_body(g, _):
        pltpu.async_copy(tb.at[idx_v.at[0, g]], rows0, sem).wait()
        pltpu.async_copy(ts.at[idx_v.at[1, g]], rows1, sem).wait()
        pltpu.async_copy(ts.at[idx_v.at[2, g]], rows2, sem).wait()
        pltpu.async_copy(ts.at[idx_v.at[3, g]], rows3, sem).wait()
        pltpu.async_copy(ts.at[idx_v.at[4, g]], rows4, sem).wait()
        pltpu.sync_copy(icont_hbm.at[wid, g], icontv)

        def slot_body(j, _):
            lu = (g * CH + j) // L
            acc = ucontv[lu] * icontv[j]
            acc += u0v[lu, pl.ds(0, 16)] * rows0[j, pl.ds(0, 16)]
            acc += u0v[lu, pl.ds(16, 16)] * rows0[j, pl.ds(16, 16)]
            acc += u0v[lu, pl.ds(32, 16)] * rows0[j, pl.ds(32, 16)]
            acc += u0v[lu, pl.ds(48, 16)] * rows0[j, pl.ds(48, 16)]
            acc += u1v[lu] * rows1[j]
            acc += u2v[lu] * rows2[j]
            acc += u3v[lu] * rows3[j]
            acc += u4v[lu] * rows4[j]
            accb[pl.ds(j * LANES, LANES)] = acc
            return 0

        lax.fori_loop(0, CH, slot_body, 0, unroll=2)

        def red_body(k, _):
            rowbase = (k * LANES + lax.iota(jnp.int32, LANES)) * LANES
            tot = jnp.zeros((LANES,), jnp.float32)
            for c in range(LANES):
                tot = tot + plsc.load_gather(accb, [rowbase + c])
            outv[pl.ds(g * CH + k * LANES, LANES)] = tot
            return 0

        lax.fori_loop(0, GR, red_body, 0)
        return 0

    lax.fori_loop(0, NCH, chunk_body, 0)

    pltpu.sync_copy(outv, out_hbm.at[wid])


@jax.jit
def kernel(user_sparse, item_sparse, user_cont, item_cont,
           user_t0, user_t1, user_t2, user_t3, user_t4,
           item_t0, item_t1, item_t2, item_t3, item_t4):
    tb = jnp.concatenate([user_t0, item_t0], axis=0)
    ts = jnp.concatenate([user_t1, user_t2, user_t3, user_t4,
                          item_t1, item_t2, item_t3, item_t4],
                         axis=0)
    uoff = jnp.array([0, 0, SMALL_VOCAB, 2 * SMALL_VOCAB, 3 * SMALL_VOCAB],
                     jnp.int32)
    ioff = jnp.array([1000000, 4 * SMALL_VOCAB, 5 * SMALL_VOCAB,
                      6 * SMALL_VOCAB, 7 * SMALL_VOCAB], jnp.int32)
    iidx = ((item_sparse + ioff).reshape(B * L, NF)
            .reshape(W, NCH, CH, NF)
            .transpose(0, 3, 1, 2))
    icont = jnp.pad(item_cont.reshape(B * L, CONT),
                    ((0, 0), (0, LANES - CONT)))
    icont = icont.reshape(W, NCH, CH, LANES)
    uidx = (user_sparse + uoff).reshape(W, UPW, NF).transpose(0, 2, 1)
    ucont = jnp.pad(user_cont, ((0, 0), (0, LANES - CONT)))
    ucont = ucont.reshape(W, UPW, LANES)

    mesh = plsc.VectorSubcoreMesh(core_axis_name="c", subcore_axis_name="s")
    run = pl.kernel(
        _sc_kernel,
        mesh=mesh,
        compiler_params=pltpu.CompilerParams(needs_layout_passes=False,
                                             use_tc_tiling_on_sc=False),
        out_type=jax.ShapeDtypeStruct((W, SPW), jnp.float32),
        scratch_types=[
            pltpu.VMEM((NF, NCH, CH), jnp.int32),
            pltpu.VMEM((NF, UPW), jnp.int32),
            pltpu.VMEM((UPW, LARGE_DIM), jnp.float32),
            pltpu.VMEM((UPW, SMALL_DIM), jnp.float32),
            pltpu.VMEM((UPW, SMALL_DIM), jnp.float32),
            pltpu.VMEM((UPW, SMALL_DIM), jnp.float32),
            pltpu.VMEM((UPW, SMALL_DIM), jnp.float32),
            pltpu.VMEM((UPW, LANES), jnp.float32),
            pltpu.VMEM((CH, LARGE_DIM), jnp.float32),
            pltpu.VMEM((CH, SMALL_DIM), jnp.float32),
            pltpu.VMEM((CH, SMALL_DIM), jnp.float32),
            pltpu.VMEM((CH, SMALL_DIM), jnp.float32),
            pltpu.VMEM((CH, SMALL_DIM), jnp.float32),
            pltpu.VMEM((CH, LANES), jnp.float32),
            pltpu.VMEM((CH * LANES,), jnp.float32),
            pltpu.VMEM((SPW,), jnp.float32),
            pltpu.SemaphoreType.DMA,
        ],
    )
    out = run(iidx, icont, uidx, ucont, tb, ts)
    return out.reshape(B, L)


# speedup vs baseline: 1.4124x; 1.4124x over previous
import jax
import jax.numpy as jnp
from jax import lax
from jax.experimental import pallas as pl
from jax.experimental.pallas import tpu as pltpu
from jax.experimental.pallas import tpu_sc as plsc

B = 4096
L = 50
NF = 5
CONT = 8
LARGE_DIM = 64
SMALL_DIM = 16
LANES = 16

NC = 2
NS = 16
W = NC * NS
UPW = B // W
SPW = UPW * L
CH = 128
NCH = SPW // CH
GR = CH // LANES


def _deinterleave(dst, src, n):
    ivec5 = lax.iota(jnp.int32, LANES) * NF
    for f in range(NF):
        for c in range(n // LANES):
            pos = ivec5 + (c * LANES * NF + f)
            dst[f, pl.ds(c * LANES, LANES)] = plsc.load_gather(src, [pos])


def _sc_kernel(isp_hbm, icf_hbm, usp_hbm, ucf_hbm,
               ut0, ut1, ut2, ut3, ut4,
               it0, it1, it2, it3, it4,
               out_hbm,
               idxr, gidx, uidxr, ugidx, icfv, ucfv,
               u0v, u1v, u2v, u3v, u4v,
               rows0, rows1, rows2, rows3, rows4,
               accb, outv, sem):
    wid = lax.axis_index("s") * NC + lax.axis_index("c")
    cmask = jnp.where(lax.iota(jnp.int32, LANES) < CONT, 1.0, 0.0)

    pltpu.sync_copy(usp_hbm.at[pl.ds(wid * UPW * NF, UPW * NF)], uidxr)
    pltpu.sync_copy(ucf_hbm.at[pl.ds(wid * UPW * CONT, UPW * CONT)],
                    ucfv.at[pl.ds(0, UPW * CONT)])
    _deinterleave(ugidx, uidxr, UPW)

    pltpu.async_copy(ut0.at[ugidx.at[0]], u0v, sem).wait()
    pltpu.async_copy(ut1.at[ugidx.at[1]], u1v, sem).wait()
    pltpu.async_copy(ut2.at[ugidx.at[2]], u2v, sem).wait()
    pltpu.async_copy(ut3.at[ugidx.at[3]], u3v, sem).wait()
    pltpu.async_copy(ut4.at[ugidx.at[4]], u4v, sem).wait()

    def chunk_body(g, _):
        base = wid * SPW + g * CH
        pltpu.sync_copy(isp_hbm.at[pl.ds(base * NF, CH * NF)], idxr)
        pltpu.sync_copy(icf_hbm.at[pl.ds(base * CONT, CH * CONT)],
                        icfv.at[pl.ds(0, CH * CONT)])
        _deinterleave(gidx, idxr, CH)

        pltpu.async_copy(it0.at[gidx.at[0]], rows0, sem).wait()
        pltpu.async_copy(it1.at[gidx.at[1]], rows1, sem).wait()
        pltpu.async_copy(it2.at[gidx.at[2]], rows2, sem).wait()
        pltpu.async_copy(it3.at[gidx.at[3]], rows3, sem).wait()
        pltpu.async_copy(it4.at[gidx.at[4]], rows4, sem).wait()

        def slot_body(j, _):
            lu = (g * CH + j) // L
            acc = (cmask * icfv[pl.ds(j * CONT, LANES)]
                   * ucfv[pl.ds(lu * CONT, LANES)])
            acc += u0v[lu, pl.ds(0, 16)] * rows0[j, pl.ds(0, 16)]
            acc += u0v[lu, pl.ds(16, 16)] * rows0[j, pl.ds(16, 16)]
            acc += u0v[lu, pl.ds(32, 16)] * rows0[j, pl.ds(32, 16)]
            acc += u0v[lu, pl.ds(48, 16)] * rows0[j, pl.ds(48, 16)]
            acc += u1v[lu] * rows1[j]
            acc += u2v[lu] * rows2[j]
            acc += u3v[lu] * rows3[j]
            acc += u4v[lu] * rows4[j]
            accb[pl.ds(j * LANES, LANES)] = acc
            return 0

        lax.fori_loop(0, CH, slot_body, 0, unroll=2)

        def red_body(k, _):
            rowbase = (k * LANES + lax.iota(jnp.int32, LANES)) * LANES
            tot = jnp.zeros((LANES,), jnp.float32)
            for c in range(LANES):
                tot = tot + plsc.load_gather(accb, [rowbase + c])
            outv[pl.ds(g * CH + k * LANES, LANES)] = tot
            return 0

        lax.fori_loop(0, GR, red_body, 0)
        return 0

    lax.fori_loop(0, NCH, chunk_body, 0)

    pltpu.sync_copy(outv, out_hbm.at[wid])


@jax.jit
def kernel(user_sparse, item_sparse, user_cont, item_cont,
           user_t0, user_t1, user_t2, user_t3, user_t4,
           item_t0, item_t1, item_t2, item_t3, item_t4):
    isp = item_sparse.reshape(B * L * NF)
    usp = user_sparse.reshape(B * NF)
    icf = item_cont.reshape(B * L * CONT)
    ucf = user_cont.reshape(B * CONT)

    mesh = plsc.VectorSubcoreMesh(core_axis_name="c", subcore_axis_name="s")
    run = pl.kernel(
        _sc_kernel,
        mesh=mesh,
        compiler_params=pltpu.CompilerParams(needs_layout_passes=False,
                                             use_tc_tiling_on_sc=False),
        out_type=jax.ShapeDtypeStruct((W, SPW), jnp.float32),
        scratch_types=[
            pltpu.VMEM((CH * NF,), jnp.int32),
            pltpu.VMEM((NF, CH), jnp.int32),
            pltpu.VMEM((UPW * NF,), jnp.int32),
            pltpu.VMEM((NF, UPW), jnp.int32),
            pltpu.VMEM((CH * CONT + LANES,), jnp.float32),
            pltpu.VMEM((UPW * CONT + LANES,), jnp.float32),
            pltpu.VMEM((UPW, LARGE_DIM), jnp.float32),
            pltpu.VMEM((UPW, SMALL_DIM), jnp.float32),
            pltpu.VMEM((UPW, SMALL_DIM), jnp.float32),
            pltpu.VMEM((UPW, SMALL_DIM), jnp.float32),
            pltpu.VMEM((UPW, SMALL_DIM), jnp.float32),
            pltpu.VMEM((CH, LARGE_DIM), jnp.float32),
            pltpu.VMEM((CH, SMALL_DIM), jnp.float32),
            pltpu.VMEM((CH, SMALL_DIM), jnp.float32),
            pltpu.VMEM((CH, SMALL_DIM), jnp.float32),
            pltpu.VMEM((CH, SMALL_DIM), jnp.float32),
            pltpu.VMEM((CH * LANES,), jnp.float32),
            pltpu.VMEM((SPW,), jnp.float32),
            pltpu.SemaphoreType.DMA,
        ],
    )
    out = run(isp, icf, usp, ucf,
              user_t0, user_t1, user_t2, user_t3, user_t4,
              item_t0, item_t1, item_t2, item_t3, item_t4)
    return out.reshape(B, L)
